# baseline (speedup 1.0000x reference)
import functools

import jax
import jax.numpy as jnp
from jax import lax
from jax.experimental import pallas as pl
from jax.experimental.pallas import tpu as pltpu
from jax.experimental.pallas import tpu_sc as plsc

N = 10000
E = 320000
D = 128
G = 64
C = 10

NC = 2
NS = 16
NW = NC * NS
CH = 128
NCHUNK = 81
EP = NW * NCHUNK * CH
NPAD = NS * 632
RPT = NPAD // NS
NBUF = 3

BM = 1000
GRID = N // BM

_mesh = plsc.VectorSubcoreMesh(
    core_axis_name="c", subcore_axis_name="s", num_cores=NC, num_subcores=NS)


@functools.partial(
    pl.kernel,
    out_type=jax.ShapeDtypeStruct((NC, NPAD, D), jnp.float32),
    mesh=_mesh,
    scratch_types=[
        [pltpu.VMEM((2, CH), jnp.int32) for _ in range(NBUF)],
        [pltpu.VMEM((CH, D), jnp.float32) for _ in range(NBUF)],
        [pltpu.SemaphoreType.DMA for _ in range(NBUF)],
        [pltpu.SemaphoreType.DMA for _ in range(NBUF)],
        pltpu.VMEM_SHARED((NPAD, D), jnp.float32),
    ],
)
def _seg_sum(y_hbm, idx_hbm, out_hbm, idxs, msgs, isems, gsems, acc_sh):
    c = lax.axis_index("c")
    s = lax.axis_index("s")
    wid = s * NC + c

    zeros16 = jnp.zeros((16,), jnp.float32)

    def _zero_row(r, carry):
        for j in range(D // 16):
            msgs[0][r, pl.ds(j * 16, 16)] = zeros16
        return carry

    lax.fori_loop(0, CH, _zero_row, 0)
    row0 = s * RPT
    for k in range(4):
        pltpu.sync_copy(msgs[0], acc_sh.at[pl.ds(row0 + k * CH, CH)])
    pltpu.sync_copy(msgs[0].at[pl.ds(0, RPT - 4 * CH)],
                    acc_sh.at[pl.ds(row0 + 4 * CH, RPT - 4 * CH)])
    plsc.subcore_barrier()

    def _fire_idx(j, b):
        pltpu.async_copy(idx_hbm.at[wid, j], idxs[b], isems[b])

    def _fire_gather(j, b):
        pltpu.make_async_copy(idx_hbm.at[wid, j], idxs[b], isems[b]).wait()
        pltpu.async_copy(y_hbm.at[idxs[b].at[0]], msgs[b], gsems[b])

    def _drain(j, b, fire_next, fire_idx):
        pltpu.make_async_copy(y_hbm.at[idxs[b].at[0]], msgs[b], gsems[b]).wait()
        if fire_next:
            _fire_gather(j + 1, (b + 1) % NBUF)
        pltpu.sync_copy(msgs[b], acc_sh.at[idxs[b].at[1]], add=True)
        if fire_idx:
            _fire_idx(j + NBUF, b)

    for b in range(NBUF):
        _fire_idx(b, b)
    _fire_gather(0, 0)

    def _ring(it, carry):
        base = it * NBUF
        for b in range(NBUF):
            _drain(base + b, b, True, True)
        return carry

    lax.fori_loop(0, (NCHUNK - NBUF) // NBUF, _ring, 0)
    for b in range(NBUF):
        j = NCHUNK - NBUF + b
        _drain(j, j % NBUF, j + 1 < NCHUNK, False)
    plsc.subcore_barrier()

    pltpu.sync_copy(acc_sh.at[pl.ds(row0, RPT)], out_hbm.at[c, pl.ds(row0, RPT)])


def _mm_nt_body(x_ref, w_ref, o_ref):
    o_ref[...] = lax.dot_general(
        x_ref[...], w_ref[...], (((1,), (1,)), ((), ())),
        preferred_element_type=jnp.float32)


def _mm_nt(x, w):
    return pl.pallas_call(
        _mm_nt_body,
        grid=(GRID,),
        in_specs=[
            pl.BlockSpec((BM, D), lambda i: (i, 0)),
            pl.BlockSpec(w.shape, lambda i: (0, 0)),
        ],
        out_specs=pl.BlockSpec((BM, D), lambda i: (i, 0)),
        out_shape=jax.ShapeDtypeStruct((N, D), jnp.float32),
    )(x, w)


def _fuse_body(relu, a0_ref, a1_ref, x_ref, wr_ref, b_ref, wn_ref, h_ref, y_ref):
    z = lax.dot_general(x_ref[...], wr_ref[...], (((1,), (1,)), ((), ())),
                        preferred_element_type=jnp.float32)
    h = a0_ref[...] + a1_ref[...] + z + b_ref[...]
    if relu:
        h = jnp.maximum(h, 0.0)
    h_ref[...] = h
    y_ref[...] = lax.dot_general(h, wn_ref[...], (((1,), (1,)), ((), ())),
                                 preferred_element_type=jnp.float32)


def _fuse(a0, a1, x, w_root, b_rel, w_next, relu):
    return pl.pallas_call(
        functools.partial(_fuse_body, relu),
        grid=(GRID,),
        in_specs=[
            pl.BlockSpec((BM, D), lambda i: (i, 0)),
            pl.BlockSpec((BM, D), lambda i: (i, 0)),
            pl.BlockSpec((BM, D), lambda i: (i, 0)),
            pl.BlockSpec((D, D), lambda i: (0, 0)),
            pl.BlockSpec((D,), lambda i: (0,)),
            pl.BlockSpec((D, D), lambda i: (0, 0)),
        ],
        out_specs=[
            pl.BlockSpec((BM, D), lambda i: (i, 0)),
            pl.BlockSpec((BM, D), lambda i: (i, 0)),
        ],
        out_shape=[
            jax.ShapeDtypeStruct((N, D), jnp.float32),
            jax.ShapeDtypeStruct((N, D), jnp.float32),
        ],
    )(a0, a1, x, w_root, b_rel, w_next)


def _final_body(a0_ref, a1_ref, x_ref, wr_ref, b_ref, bat_ref, wl_ref, bl_ref,
                o_ref, pool_ref, cnt_ref):
    i = pl.program_id(0)

    @pl.when(i == 0)
    def _init():
        pool_ref[...] = jnp.zeros_like(pool_ref)
        cnt_ref[...] = jnp.zeros_like(cnt_ref)

    z = lax.dot_general(x_ref[...], wr_ref[...], (((1,), (1,)), ((), ())),
                        preferred_element_type=jnp.float32)
    h = a0_ref[...] + a1_ref[...] + z + b_ref[...]
    gids = bat_ref[...]
    iot = lax.broadcasted_iota(jnp.int32, (BM, G), 1)
    onehot = jnp.where(gids == iot, 1.0, 0.0)
    pool_ref[...] += lax.dot_general(
        onehot, h, (((0,), (0,)), ((), ())), preferred_element_type=jnp.float32)
    cnt_ref[...] += lax.dot_general(
        onehot, jnp.ones((BM, D), jnp.float32), (((0,), (0,)), ((), ())),
        preferred_element_type=jnp.float32)

    @pl.when(i == GRID - 1)
    def _done():
        pooled = pool_ref[...] / jnp.maximum(cnt_ref[...], 1.0)
        o_ref[...] = lax.dot_general(
            pooled, wl_ref[...], (((1,), (1,)), ((), ())),
            preferred_element_type=jnp.float32) + bl_ref[...]


def _final(a0, a1, x, w_root, b_rel, batch2d, w_lin, b_lin):
    return pl.pallas_call(
        _final_body,
        grid=(GRID,),
        in_specs=[
            pl.BlockSpec((BM, D), lambda i: (i, 0)),
            pl.BlockSpec((BM, D), lambda i: (i, 0)),
            pl.BlockSpec((BM, D), lambda i: (i, 0)),
            pl.BlockSpec((D, D), lambda i: (0, 0)),
            pl.BlockSpec((D,), lambda i: (0,)),
            pl.BlockSpec((BM, 1), lambda i: (i, 0)),
            pl.BlockSpec((C, D), lambda i: (0, 0)),
            pl.BlockSpec((C,), lambda i: (0,)),
        ],
        out_specs=pl.BlockSpec((G, C), lambda i: (0, 0)),
        out_shape=jax.ShapeDtypeStruct((G, C), jnp.float32),
        scratch_shapes=[
            pltpu.VMEM((G, D), jnp.float32),
            pltpu.VMEM((G, D), jnp.float32),
        ],
    )(a0, a1, x, w_root, b_rel, batch2d, w_lin, b_lin)


def kernel(x, edge_index, batch,
           W1_rel, b1_rel, W1_root,
           W2_rel, b2_rel, W2_root,
           W3_rel, b3_rel, W3_root,
           W_lin, b_lin):
    npad = EP - E
    srcp = jnp.concatenate(
        [edge_index[0], jnp.zeros((npad,), jnp.int32)]).reshape(NW, NCHUNK, 1, CH)
    dstp = jnp.concatenate(
        [edge_index[1], jnp.full((npad,), N, jnp.int32)]).reshape(NW, NCHUNK, 1, CH)
    idx = jnp.concatenate([srcp, dstp], axis=2)
    batch2d = batch.reshape(N, 1)

    y1 = _mm_nt(x, W1_rel)
    a1 = _seg_sum(y1, idx)
    h1, y2 = _fuse(a1[0, :N], a1[1, :N], x, W1_root, b1_rel, W2_rel, relu=True)
    a2 = _seg_sum(y2, idx)
    h2, y3 = _fuse(a2[0, :N], a2[1, :N], h1, W2_root, b2_rel, W3_rel, relu=True)
    a3 = _seg_sum(y3, idx)
    return _final(a3[0, :N], a3[1, :N], h2, W3_root, b3_rel, batch2d, W_lin, b_lin)


# speedup vs baseline: 1.7700x; 1.7700x over previous
import functools

import jax
import jax.numpy as jnp
from jax import lax
from jax.experimental import pallas as pl
from jax.experimental.pallas import tpu as pltpu
from jax.experimental.pallas import tpu_sc as plsc

N = 10000
E = 320000
D = 128
G = 64
C = 10

NC = 2
NS = 16
NW = NC * NS
CH = 96
NCHUNK = 106
EW = NCHUNK * CH
EP = NW * EW
NPAD = NS * 632
RPT = NPAD // NS
MSG_BYTES = CH * D * 4

BM = 1000
GRID = N // BM

_mesh = plsc.VectorSubcoreMesh(
    core_axis_name="c", subcore_axis_name="s", num_cores=NC, num_subcores=NS)


@functools.partial(
    pl.kernel,
    out_type=jax.ShapeDtypeStruct((NC, NPAD, D), jnp.float32),
    mesh=_mesh,
    scratch_types=[
        pltpu.VMEM((EW,), jnp.int32),
        pltpu.VMEM((NCHUNK, CH), jnp.int32),
        [pltpu.VMEM((CH, D), jnp.float32) for _ in range(2)],
        [pltpu.SemaphoreType.DMA for _ in range(2)],
        [pltpu.SemaphoreType.DMA for _ in range(2)],
        pltpu.VMEM_SHARED((NPAD, D), jnp.float32),
    ],
)
def _seg_sum(y_hbm, src_hbm, dst_hbm, out_hbm, src_v, dst_v, msgs, gsems, ssems,
             acc_sh):
    c = lax.axis_index("c")
    s = lax.axis_index("s")
    wid = s * NC + c

    zeros16 = jnp.zeros((16,), jnp.float32)

    def _zero_row(r, carry):
        for j in range(D // 16):
            msgs[0][r, pl.ds(j * 16, 16)] = zeros16
        return carry

    lax.fori_loop(0, CH, _zero_row, 0)
    row0 = s * RPT
    for k in range(RPT // CH):
        pltpu.sync_copy(msgs[0], acc_sh.at[pl.ds(row0 + k * CH, CH)])
    pltpu.sync_copy(msgs[0].at[pl.ds(0, RPT % CH)],
                    acc_sh.at[pl.ds(row0 + RPT - RPT % CH, RPT % CH)])
    plsc.subcore_barrier()

    pltpu.sync_copy(src_hbm.at[wid], src_v)
    pltpu.sync_copy(dst_hbm.at[wid], dst_v)

    def _fire_gather(j, b):
        base = pl.multiple_of(j * CH, CH)
        pltpu.async_copy(y_hbm.at[src_v.at[pl.ds(base, CH)]], msgs[b], gsems[b])

    def _sem_wait(sem, b):
        pltpu.make_async_copy(y_hbm.at[pl.ds(0, CH)], msgs[b], sem).wait()

    def _visit(j, b, fire_next):
        _sem_wait(gsems[b], b)
        pltpu.async_copy(msgs[b], acc_sh.at[dst_v.at[j]], ssems[b], add=True)
        if fire_next:
            _sem_wait(ssems[1 - b], 1 - b)
            _fire_gather(j + 1, 1 - b)

    _fire_gather(0, 0)
    _fire_gather(1, 1)
    _visit(0, 0, False)
    _visit(1, 1, True)

    def _ring(it, carry):
        j0 = 2 + 2 * it
        _visit(j0, 0, True)
        _visit(j0 + 1, 1, True)
        return carry

    lax.fori_loop(0, (NCHUNK - 4) // 2, _ring, 0)
    _visit(NCHUNK - 2, 0, True)
    _visit(NCHUNK - 1, 1, False)
    _sem_wait(ssems[0], 0)
    _sem_wait(ssems[1], 1)
    plsc.subcore_barrier()

    pltpu.sync_copy(acc_sh.at[pl.ds(row0, RPT)], out_hbm.at[c, pl.ds(row0, RPT)])


def _mm_nt_body(x_ref, w_ref, o_ref):
    o_ref[...] = lax.dot_general(
        x_ref[...], w_ref[...], (((1,), (1,)), ((), ())),
        preferred_element_type=jnp.float32)


def _mm_nt(x, w):
    return pl.pallas_call(
        _mm_nt_body,
        grid=(GRID,),
        in_specs=[
            pl.BlockSpec((BM, D), lambda i: (i, 0)),
            pl.BlockSpec(w.shape, lambda i: (0, 0)),
        ],
        out_specs=pl.BlockSpec((BM, D), lambda i: (i, 0)),
        out_shape=jax.ShapeDtypeStruct((N, D), jnp.float32),
    )(x, w)


def _fuse_body(relu, a0_ref, a1_ref, x_ref, wr_ref, b_ref, wn_ref, h_ref, y_ref):
    z = lax.dot_general(x_ref[...], wr_ref[...], (((1,), (1,)), ((), ())),
                        preferred_element_type=jnp.float32)
    h = a0_ref[...] + a1_ref[...] + z + b_ref[...]
    if relu:
        h = jnp.maximum(h, 0.0)
    h_ref[...] = h
    y_ref[...] = lax.dot_general(h, wn_ref[...], (((1,), (1,)), ((), ())),
                                 preferred_element_type=jnp.float32)


def _fuse(a0, a1, x, w_root, b_rel, w_next, relu):
    return pl.pallas_call(
        functools.partial(_fuse_body, relu),
        grid=(GRID,),
        in_specs=[
            pl.BlockSpec((BM, D), lambda i: (i, 0)),
            pl.BlockSpec((BM, D), lambda i: (i, 0)),
            pl.BlockSpec((BM, D), lambda i: (i, 0)),
            pl.BlockSpec((D, D), lambda i: (0, 0)),
            pl.BlockSpec((D,), lambda i: (0,)),
            pl.BlockSpec((D, D), lambda i: (0, 0)),
        ],
        out_specs=[
            pl.BlockSpec((BM, D), lambda i: (i, 0)),
            pl.BlockSpec((BM, D), lambda i: (i, 0)),
        ],
        out_shape=[
            jax.ShapeDtypeStruct((N, D), jnp.float32),
            jax.ShapeDtypeStruct((N, D), jnp.float32),
        ],
    )(a0, a1, x, w_root, b_rel, w_next)


def _final_body(a0_ref, a1_ref, x_ref, wr_ref, b_ref, bat_ref, wl_ref, bl_ref,
                o_ref, pool_ref, cnt_ref):
    i = pl.program_id(0)

    @pl.when(i == 0)
    def _init():
        pool_ref[...] = jnp.zeros_like(pool_ref)
        cnt_ref[...] = jnp.zeros_like(cnt_ref)

    z = lax.dot_general(x_ref[...], wr_ref[...], (((1,), (1,)), ((), ())),
                        preferred_element_type=jnp.float32)
    h = a0_ref[...] + a1_ref[...] + z + b_ref[...]
    gids = bat_ref[...]
    iot = lax.broadcasted_iota(jnp.int32, (BM, G), 1)
    onehot = jnp.where(gids == iot, 1.0, 0.0)
    pool_ref[...] += lax.dot_general(
        onehot, h, (((0,), (0,)), ((), ())), preferred_element_type=jnp.float32)
    cnt_ref[...] += lax.dot_general(
        onehot, jnp.ones((BM, D), jnp.float32), (((0,), (0,)), ((), ())),
        preferred_element_type=jnp.float32)

    @pl.when(i == GRID - 1)
    def _done():
        pooled = pool_ref[...] / jnp.maximum(cnt_ref[...], 1.0)
        o_ref[...] = lax.dot_general(
            pooled, wl_ref[...], (((1,), (1,)), ((), ())),
            preferred_element_type=jnp.float32) + bl_ref[...]


def _final(a0, a1, x, w_root, b_rel, batch2d, w_lin, b_lin):
    return pl.pallas_call(
        _final_body,
        grid=(GRID,),
        in_specs=[
            pl.BlockSpec((BM, D), lambda i: (i, 0)),
            pl.BlockSpec((BM, D), lambda i: (i, 0)),
            pl.BlockSpec((BM, D), lambda i: (i, 0)),
            pl.BlockSpec((D, D), lambda i: (0, 0)),
            pl.BlockSpec((D,), lambda i: (0,)),
            pl.BlockSpec((BM, 1), lambda i: (i, 0)),
            pl.BlockSpec((C, D), lambda i: (0, 0)),
            pl.BlockSpec((C,), lambda i: (0,)),
        ],
        out_specs=pl.BlockSpec((G, C), lambda i: (0, 0)),
        out_shape=jax.ShapeDtypeStruct((G, C), jnp.float32),
        scratch_shapes=[
            pltpu.VMEM((G, D), jnp.float32),
            pltpu.VMEM((G, D), jnp.float32),
        ],
    )(a0, a1, x, w_root, b_rel, batch2d, w_lin, b_lin)


def kernel(x, edge_index, batch,
           W1_rel, b1_rel, W1_root,
           W2_rel, b2_rel, W2_root,
           W3_rel, b3_rel, W3_root,
           W_lin, b_lin):
    npad = EP - E
    src = jnp.concatenate(
        [edge_index[0], jnp.zeros((npad,), jnp.int32)]).reshape(NW, EW)
    dst = jnp.concatenate(
        [edge_index[1], jnp.full((npad,), N, jnp.int32)]).reshape(NW, NCHUNK, CH)
    batch2d = batch.reshape(N, 1)

    y1 = _mm_nt(x, W1_rel)
    a1 = _seg_sum(y1, src, dst)
    h1, y2 = _fuse(a1[0, :N], a1[1, :N], x, W1_root, b1_rel, W2_rel, relu=True)
    a2 = _seg_sum(y2, src, dst)
    h2, y3 = _fuse(a2[0, :N], a2[1, :N], h1, W2_root, b2_rel, W3_rel, relu=True)
    a3 = _seg_sum(y3, src, dst)
    return _final(a3[0, :N], a3[1, :N], h2, W3_root, b3_rel, batch2d, W_lin, b_lin)


# speedup vs baseline: 4.2538x; 2.4032x over previous
import functools

import jax
import jax.numpy as jnp
from jax import lax
from jax.experimental import pallas as pl
from jax.experimental.pallas import tpu as pltpu
from jax.experimental.pallas import tpu_sc as plsc

N = 10000
E = 320000
D = 128
G = 64
C = 10

NC = 2
NS = 16
NW = NC * NS
CH = 96
NCHUNK = 106
EW = NCHUNK * CH
EP = NW * EW
NPAD = NS * 632
RPT = NPAD // NS
MSG_BYTES = CH * D * 4

BM = 1000
GRID = N // BM

_mesh = plsc.VectorSubcoreMesh(
    core_axis_name="c", subcore_axis_name="s", num_cores=NC, num_subcores=NS)


@functools.partial(
    pl.kernel,
    out_type=jax.ShapeDtypeStruct((NC, NPAD, D), jnp.float32),
    mesh=_mesh,
    scratch_types=[
        pltpu.VMEM((EW,), jnp.int32),
        pltpu.VMEM((NCHUNK, CH), jnp.int32),
        [pltpu.VMEM((CH, D), jnp.float32) for _ in range(2)],
        [pltpu.SemaphoreType.DMA for _ in range(2)],
        [pltpu.SemaphoreType.DMA for _ in range(2)],
        pltpu.VMEM_SHARED((NPAD, D), jnp.float32),
    ],
)
def _seg_sum(y_hbm, src_hbm, dst_hbm, out_hbm, src_v, dst_v, msgs, gsems, ssems,
             acc_sh):
    c = lax.axis_index("c")
    s = lax.axis_index("s")
    wid = s * NC + c

    zeros16 = jnp.zeros((16,), jnp.float32)

    def _zero_row(r, carry):
        for j in range(D // 16):
            msgs[0][r, pl.ds(j * 16, 16)] = zeros16
        return carry

    lax.fori_loop(0, CH, _zero_row, 0)
    row0 = s * RPT
    for k in range(RPT // CH):
        pltpu.sync_copy(msgs[0], acc_sh.at[pl.ds(row0 + k * CH, CH)])
    pltpu.sync_copy(msgs[0].at[pl.ds(0, RPT % CH)],
                    acc_sh.at[pl.ds(row0 + RPT - RPT % CH, RPT % CH)])
    plsc.subcore_barrier()

    pltpu.sync_copy(src_hbm.at[wid], src_v)
    pltpu.sync_copy(dst_hbm.at[wid], dst_v)

    def _fire_gather(j, b):
        base = pl.multiple_of(j * CH, CH)
        pltpu.async_copy(y_hbm.at[src_v.at[pl.ds(base, CH)]], msgs[b], gsems[b])

    def _sem_wait(sem, b):
        pltpu.make_async_copy(y_hbm.at[pl.ds(0, CH)], msgs[b], sem).wait()

    def _visit(j, b, fire_next):
        _sem_wait(gsems[b], b)
        pltpu.async_copy(msgs[b], acc_sh.at[dst_v.at[j]], ssems[b], add=True)
        if fire_next:
            _sem_wait(ssems[1 - b], 1 - b)
            _fire_gather(j + 1, 1 - b)

    _fire_gather(0, 0)
    _fire_gather(1, 1)
    _visit(0, 0, False)
    _visit(1, 1, True)

    def _ring(it, carry):
        j0 = 2 + 2 * it
        _visit(j0, 0, True)
        _visit(j0 + 1, 1, True)
        return carry

    lax.fori_loop(0, (NCHUNK - 4) // 2, _ring, 0)
    _visit(NCHUNK - 2, 0, True)
    _visit(NCHUNK - 1, 1, False)
    _sem_wait(ssems[0], 0)
    _sem_wait(ssems[1], 1)
    plsc.subcore_barrier()

    pltpu.sync_copy(acc_sh.at[pl.ds(row0, RPT)], out_hbm.at[c, pl.ds(row0, RPT)])


def _mm_nt_body(x_ref, w_ref, o_ref):
    o_ref[...] = lax.dot_general(
        x_ref[...], w_ref[...], (((1,), (1,)), ((), ())),
        preferred_element_type=jnp.float32)


def _mm_nt(x, w):
    return pl.pallas_call(
        _mm_nt_body,
        grid=(GRID,),
        in_specs=[
            pl.BlockSpec((BM, D), lambda i: (i, 0)),
            pl.BlockSpec(w.shape, lambda i: (0, 0)),
        ],
        out_specs=pl.BlockSpec((BM, D), lambda i: (i, 0)),
        out_shape=jax.ShapeDtypeStruct((N, D), jnp.float32),
    )(x, w)


def _fuse_body(relu, a0_ref, a1_ref, x_ref, wr_ref, b_ref, wn_ref, h_ref, y_ref):
    z = lax.dot_general(x_ref[...], wr_ref[...], (((1,), (1,)), ((), ())),
                        preferred_element_type=jnp.float32)
    h = a0_ref[...] + a1_ref[...] + z + b_ref[...]
    if relu:
        h = jnp.maximum(h, 0.0)
    h_ref[...] = h
    y_ref[...] = lax.dot_general(h, wn_ref[...], (((1,), (1,)), ((), ())),
                                 preferred_element_type=jnp.float32)


def _fuse(a0, a1, x, w_root, b_rel, w_next, relu):
    return pl.pallas_call(
        functools.partial(_fuse_body, relu),
        grid=(GRID,),
        in_specs=[
            pl.BlockSpec((BM, D), lambda i: (i, 0)),
            pl.BlockSpec((BM, D), lambda i: (i, 0)),
            pl.BlockSpec((BM, D), lambda i: (i, 0)),
            pl.BlockSpec((D, D), lambda i: (0, 0)),
            pl.BlockSpec((D,), lambda i: (0,)),
            pl.BlockSpec((D, D), lambda i: (0, 0)),
        ],
        out_specs=[
            pl.BlockSpec((BM, D), lambda i: (i, 0)),
            pl.BlockSpec((BM, D), lambda i: (i, 0)),
        ],
        out_shape=[
            jax.ShapeDtypeStruct((N, D), jnp.float32),
            jax.ShapeDtypeStruct((N, D), jnp.float32),
        ],
    )(a0, a1, x, w_root, b_rel, w_next)


def _final_body(a0_ref, a1_ref, x_ref, wr_ref, b_ref, bat_ref, wl_ref, bl_ref,
                o_ref, pool_ref, cnt_ref):
    i = pl.program_id(0)

    @pl.when(i == 0)
    def _init():
        pool_ref[...] = jnp.zeros_like(pool_ref)
        cnt_ref[...] = jnp.zeros_like(cnt_ref)

    z = lax.dot_general(x_ref[...], wr_ref[...], (((1,), (1,)), ((), ())),
                        preferred_element_type=jnp.float32)
    h = a0_ref[...] + a1_ref[...] + z + b_ref[...]
    gids = bat_ref[...]
    iot = lax.broadcasted_iota(jnp.int32, (BM, G), 1)
    onehot = jnp.where(gids == iot, 1.0, 0.0)
    pool_ref[...] += lax.dot_general(
        onehot, h, (((0,), (0,)), ((), ())), preferred_element_type=jnp.float32)
    cnt_ref[...] += lax.dot_general(
        onehot, jnp.ones((BM, D), jnp.float32), (((0,), (0,)), ((), ())),
        preferred_element_type=jnp.float32)

    @pl.when(i == GRID - 1)
    def _done():
        pooled = pool_ref[...] / jnp.maximum(cnt_ref[...], 1.0)
        o_ref[...] = lax.dot_general(
            pooled, wl_ref[...], (((1,), (1,)), ((), ())),
            preferred_element_type=jnp.float32) + bl_ref[...]


def _final(a0, a1, x, w_root, b_rel, batch2d, w_lin, b_lin):
    return pl.pallas_call(
        _final_body,
        grid=(GRID,),
        in_specs=[
            pl.BlockSpec((BM, D), lambda i: (i, 0)),
            pl.BlockSpec((BM, D), lambda i: (i, 0)),
            pl.BlockSpec((BM, D), lambda i: (i, 0)),
            pl.BlockSpec((D, D), lambda i: (0, 0)),
            pl.BlockSpec((D,), lambda i: (0,)),
            pl.BlockSpec((BM, 1), lambda i: (i, 0)),
            pl.BlockSpec((C, D), lambda i: (0, 0)),
            pl.BlockSpec((C,), lambda i: (0,)),
        ],
        out_specs=pl.BlockSpec((G, C), lambda i: (0, 0)),
        out_shape=jax.ShapeDtypeStruct((G, C), jnp.float32),
        scratch_shapes=[
            pltpu.VMEM((G, D), jnp.float32),
            pltpu.VMEM((G, D), jnp.float32),
        ],
    )(a0, a1, x, w_root, b_rel, batch2d, w_lin, b_lin)


def kernel(x, edge_index, batch,
           W1_rel, b1_rel, W1_root,
           W2_rel, b2_rel, W2_root,
           W3_rel, b3_rel, W3_root,
           W_lin, b_lin):
    npad = EP - E
    pad_src = jnp.arange(npad, dtype=jnp.int32) % N
    pad_dst = N + (jnp.arange(npad, dtype=jnp.int32) % (NPAD - N))
    src = jnp.concatenate([edge_index[0], pad_src]).reshape(NW, EW)
    dst = jnp.concatenate([edge_index[1], pad_dst]).reshape(NW, NCHUNK, CH)
    batch2d = batch.reshape(N, 1)

    y1 = _mm_nt(x, W1_rel)
    a1 = _seg_sum(y1, src, dst)
    h1, y2 = _fuse(a1[0, :N], a1[1, :N], x, W1_root, b1_rel, W2_rel, relu=True)
    a2 = _seg_sum(y2, src, dst)
    h2, y3 = _fuse(a2[0, :N], a2[1, :N], h1, W2_root, b2_rel, W3_rel, relu=True)
    a3 = _seg_sum(y3, src, dst)
    return _final(a3[0, :N], a3[1, :N], h2, W3_root, b3_rel, batch2d, W_lin, b_lin)


# speedup vs baseline: 4.3154x; 1.0145x over previous
import functools

import jax
import jax.numpy as jnp
from jax import lax
from jax.experimental import pallas as pl
from jax.experimental.pallas import tpu as pltpu
from jax.experimental.pallas import tpu_sc as plsc

N = 10000
E = 320000
D = 128
G = 64
C = 10

NC = 2
NS = 16
NW = NC * NS
CH = 96
NCHUNK = 106
EW = NCHUNK * CH
EP = NW * EW
NPAD = NS * 632
RPT = NPAD // NS
MSG_BYTES = CH * D * 4

BM = 2000
GRID = N // BM

_mesh = plsc.VectorSubcoreMesh(
    core_axis_name="c", subcore_axis_name="s", num_cores=NC, num_subcores=NS)


@functools.partial(
    pl.kernel,
    out_type=jax.ShapeDtypeStruct((NC, NPAD, D), jnp.float32),
    mesh=_mesh,
    scratch_types=[
        pltpu.VMEM((EW,), jnp.int32),
        pltpu.VMEM((NCHUNK, CH), jnp.int32),
        [pltpu.VMEM((CH, D), jnp.float32) for _ in range(2)],
        [pltpu.SemaphoreType.DMA for _ in range(2)],
        [pltpu.SemaphoreType.DMA for _ in range(2)],
        pltpu.VMEM_SHARED((NPAD, D), jnp.float32),
    ],
)
def _seg_sum(y_hbm, src_hbm, dst_hbm, out_hbm, src_v, dst_v, msgs, gsems, ssems,
             acc_sh):
    c = lax.axis_index("c")
    s = lax.axis_index("s")
    wid = s * NC + c

    zeros16 = jnp.zeros((16,), jnp.float32)

    def _zero_row(r, carry):
        for j in range(D // 16):
            msgs[0][r, pl.ds(j * 16, 16)] = zeros16
        return carry

    lax.fori_loop(0, CH, _zero_row, 0)
    row0 = s * RPT
    for k in range(RPT // CH):
        pltpu.sync_copy(msgs[0], acc_sh.at[pl.ds(row0 + k * CH, CH)])
    pltpu.sync_copy(msgs[0].at[pl.ds(0, RPT % CH)],
                    acc_sh.at[pl.ds(row0 + RPT - RPT % CH, RPT % CH)])
    plsc.subcore_barrier()

    pltpu.sync_copy(src_hbm.at[wid], src_v)
    pltpu.sync_copy(dst_hbm.at[wid], dst_v)

    def _fire_gather(j, b):
        base = pl.multiple_of(j * CH, CH)
        pltpu.async_copy(y_hbm.at[src_v.at[pl.ds(base, CH)]], msgs[b], gsems[b])

    def _sem_wait(sem, b):
        pltpu.make_async_copy(y_hbm.at[pl.ds(0, CH)], msgs[b], sem).wait()

    def _visit(j, b, fire_next):
        _sem_wait(gsems[b], b)
        pltpu.async_copy(msgs[b], acc_sh.at[dst_v.at[j]], ssems[b], add=True)
        if fire_next:
            _sem_wait(ssems[1 - b], 1 - b)
            _fire_gather(j + 1, 1 - b)

    _fire_gather(0, 0)
    _fire_gather(1, 1)
    _visit(0, 0, False)
    _visit(1, 1, True)

    def _ring(it, carry):
        j0 = 2 + 2 * it
        _visit(j0, 0, True)
        _visit(j0 + 1, 1, True)
        return carry

    lax.fori_loop(0, (NCHUNK - 4) // 2, _ring, 0)
    _visit(NCHUNK - 2, 0, True)
    _visit(NCHUNK - 1, 1, False)
    _sem_wait(ssems[0], 0)
    _sem_wait(ssems[1], 1)
    plsc.subcore_barrier()

    pltpu.sync_copy(acc_sh.at[pl.ds(row0, RPT)], out_hbm.at[c, pl.ds(row0, RPT)])


def _mm_nt_body(x_ref, w_ref, o_ref):
    o_ref[...] = lax.dot_general(
        x_ref[...], w_ref[...], (((1,), (1,)), ((), ())),
        preferred_element_type=jnp.float32)


def _mm_nt(x, w):
    return pl.pallas_call(
        _mm_nt_body,
        grid=(GRID,),
        in_specs=[
            pl.BlockSpec((BM, D), lambda i: (i, 0)),
            pl.BlockSpec(w.shape, lambda i: (0, 0)),
        ],
        out_specs=pl.BlockSpec((BM, D), lambda i: (i, 0)),
        out_shape=jax.ShapeDtypeStruct((N, D), jnp.float32),
    )(x, w)


def _zlin_body(x_ref, w_ref, b_ref, o_ref):
    o_ref[...] = lax.dot_general(
        x_ref[...], w_ref[...], (((1,), (1,)), ((), ())),
        preferred_element_type=jnp.float32) + b_ref[...]


def _zlin(x, w, b):
    return pl.pallas_call(
        _zlin_body,
        grid=(GRID,),
        in_specs=[
            pl.BlockSpec((BM, D), lambda i: (i, 0)),
            pl.BlockSpec((D, D), lambda i: (0, 0)),
            pl.BlockSpec((D,), lambda i: (0,)),
        ],
        out_specs=pl.BlockSpec((BM, D), lambda i: (i, 0)),
        out_shape=jax.ShapeDtypeStruct((N, D), jnp.float32),
    )(x, w, b)


def _comb_body(relu, a0_ref, a1_ref, z_ref, wn_ref, h_ref, y_ref):
    h = a0_ref[...] + a1_ref[...] + z_ref[...]
    if relu:
        h = jnp.maximum(h, 0.0)
    h_ref[...] = h
    y_ref[...] = lax.dot_general(h, wn_ref[...], (((1,), (1,)), ((), ())),
                                 preferred_element_type=jnp.float32)


def _comb(a0, a1, z, w_next, relu):
    return pl.pallas_call(
        functools.partial(_comb_body, relu),
        grid=(GRID,),
        in_specs=[
            pl.BlockSpec((BM, D), lambda i: (i, 0)),
            pl.BlockSpec((BM, D), lambda i: (i, 0)),
            pl.BlockSpec((BM, D), lambda i: (i, 0)),
            pl.BlockSpec((D, D), lambda i: (0, 0)),
        ],
        out_specs=[
            pl.BlockSpec((BM, D), lambda i: (i, 0)),
            pl.BlockSpec((BM, D), lambda i: (i, 0)),
        ],
        out_shape=[
            jax.ShapeDtypeStruct((N, D), jnp.float32),
            jax.ShapeDtypeStruct((N, D), jnp.float32),
        ],
    )(a0, a1, z, w_next)


def _final_body(a0_ref, a1_ref, z_ref, bat_ref, wl_ref, bl_ref,
                o_ref, pool_ref, cnt_ref):
    i = pl.program_id(0)

    @pl.when(i == 0)
    def _init():
        pool_ref[...] = jnp.zeros_like(pool_ref)
        cnt_ref[...] = jnp.zeros_like(cnt_ref)

    h = a0_ref[...] + a1_ref[...] + z_ref[...]
    gids = bat_ref[...]
    iot = lax.broadcasted_iota(jnp.int32, (BM, G), 1)
    onehot = jnp.where(gids == iot, 1.0, 0.0)
    pool_ref[...] += lax.dot_general(
        onehot, h, (((0,), (0,)), ((), ())), preferred_element_type=jnp.float32)
    cnt_ref[...] += lax.dot_general(
        onehot, jnp.ones((BM, D), jnp.float32), (((0,), (0,)), ((), ())),
        preferred_element_type=jnp.float32)

    @pl.when(i == GRID - 1)
    def _done():
        pooled = pool_ref[...] / jnp.maximum(cnt_ref[...], 1.0)
        o_ref[...] = lax.dot_general(
            pooled, wl_ref[...], (((1,), (1,)), ((), ())),
            preferred_element_type=jnp.float32) + bl_ref[...]


def _final(a0, a1, z, batch2d, w_lin, b_lin):
    return pl.pallas_call(
        _final_body,
        grid=(GRID,),
        in_specs=[
            pl.BlockSpec((BM, D), lambda i: (i, 0)),
            pl.BlockSpec((BM, D), lambda i: (i, 0)),
            pl.BlockSpec((BM, D), lambda i: (i, 0)),
            pl.BlockSpec((BM, 1), lambda i: (i, 0)),
            pl.BlockSpec((C, D), lambda i: (0, 0)),
            pl.BlockSpec((C,), lambda i: (0,)),
        ],
        out_specs=pl.BlockSpec((G, C), lambda i: (0, 0)),
        out_shape=jax.ShapeDtypeStruct((G, C), jnp.float32),
        scratch_shapes=[
            pltpu.VMEM((G, D), jnp.float32),
            pltpu.VMEM((G, D), jnp.float32),
        ],
    )(a0, a1, z, batch2d, w_lin, b_lin)


def kernel(x, edge_index, batch,
           W1_rel, b1_rel, W1_root,
           W2_rel, b2_rel, W2_root,
           W3_rel, b3_rel, W3_root,
           W_lin, b_lin):
    npad = EP - E
    pad_src = jnp.arange(npad, dtype=jnp.int32) % N
    pad_dst = N + (jnp.arange(npad, dtype=jnp.int32) % (NPAD - N))
    src = jnp.concatenate([edge_index[0], pad_src]).reshape(NW, EW)
    dst = jnp.concatenate([edge_index[1], pad_dst]).reshape(NW, NCHUNK, CH)
    batch2d = batch.reshape(N, 1)

    y1 = _mm_nt(x, W1_rel)
    a1 = _seg_sum(y1, src, dst)
    z1 = _zlin(x, W1_root, b1_rel)
    h1, y2 = _comb(a1[0, :N], a1[1, :N], z1, W2_rel, relu=True)
    a2 = _seg_sum(y2, src, dst)
    z2 = _zlin(h1, W2_root, b2_rel)
    h2, y3 = _comb(a2[0, :N], a2[1, :N], z2, W3_rel, relu=True)
    a3 = _seg_sum(y3, src, dst)
    z3 = _zlin(h2, W3_root, b3_rel)
    return _final(a3[0, :N], a3[1, :N], z3, batch2d, W_lin, b_lin)


# speedup vs baseline: 5.2892x; 1.2257x over previous
import functools

import jax
import jax.numpy as jnp
from jax import lax
from jax.experimental import pallas as pl
from jax.experimental.pallas import tpu as pltpu
from jax.experimental.pallas import tpu_sc as plsc

N = 10000
E = 320000
D = 128
G = 64
C = 10

NC = 2
NS = 16
NW = NC * NS
CH = 96
NCHUNK = 106
EW = NCHUNK * CH
EP = NW * EW
NPAD = NS * 632
RPT = NPAD // NS
MSG_BYTES = CH * D * 4

BM = 2000
GRID = N // BM

_mesh = plsc.VectorSubcoreMesh(
    core_axis_name="c", subcore_axis_name="s", num_cores=NC, num_subcores=NS)


@functools.partial(
    pl.kernel,
    out_type=jax.ShapeDtypeStruct((NC, NPAD, D), jnp.float32),
    mesh=_mesh,
    scratch_types=[
        pltpu.VMEM((EW,), jnp.int32),
        pltpu.VMEM((NCHUNK, CH), jnp.int32),
        [pltpu.VMEM((CH, D), jnp.float32) for _ in range(2)],
        [pltpu.SemaphoreType.DMA for _ in range(2)],
        [pltpu.SemaphoreType.DMA for _ in range(2)],
        pltpu.VMEM_SHARED((NPAD, D), jnp.float32),
    ],
)
def _seg_sum(y_hbm, src_hbm, dst_hbm, out_hbm, src_v, dst_v, msgs, gsems, ssems,
             acc_sh):
    c = lax.axis_index("c")
    s = lax.axis_index("s")
    wid = s * NC + c

    zeros16 = jnp.zeros((16,), jnp.float32)

    def _zero_row(r, carry):
        for j in range(D // 16):
            msgs[0][r, pl.ds(j * 16, 16)] = zeros16
        return carry

    lax.fori_loop(0, CH, _zero_row, 0)
    row0 = s * RPT
    for k in range(RPT // CH):
        pltpu.sync_copy(msgs[0], acc_sh.at[pl.ds(row0 + k * CH, CH)])
    pltpu.sync_copy(msgs[0].at[pl.ds(0, RPT % CH)],
                    acc_sh.at[pl.ds(row0 + RPT - RPT % CH, RPT % CH)])
    plsc.subcore_barrier()

    pltpu.sync_copy(src_hbm.at[wid], src_v)
    pltpu.sync_copy(dst_hbm.at[wid], dst_v)

    def _fire_gather(j, b):
        base = pl.multiple_of(j * CH, CH)
        pltpu.async_copy(y_hbm.at[src_v.at[pl.ds(base, CH)]], msgs[b], gsems[b])

    def _sem_wait(sem, b):
        pltpu.make_async_copy(y_hbm.at[pl.ds(0, CH)], msgs[b], sem).wait()

    def _visit(j, b, mode):
        if mode != 2:
            if mode == 1:
                _sem_wait(ssems[1 - b], 1 - b)
            _fire_gather(j + 1, 1 - b)
        _sem_wait(gsems[b], b)
        pltpu.async_copy(msgs[b], acc_sh.at[dst_v.at[j]], ssems[b], add=True)

    _fire_gather(0, 0)
    _visit(0, 0, 0)

    def _ring(it, carry):
        j0 = 1 + 2 * it
        _visit(j0, 1, 1)
        _visit(j0 + 1, 0, 1)
        return carry

    lax.fori_loop(0, (NCHUNK - 2) // 2, _ring, 0)
    _visit(NCHUNK - 1, 1, 2)
    _sem_wait(ssems[0], 0)
    _sem_wait(ssems[1], 1)
    plsc.subcore_barrier()

    pltpu.sync_copy(acc_sh.at[pl.ds(row0, RPT)], out_hbm.at[c, pl.ds(row0, RPT)])


def _mm_nt_body(x_ref, w_ref, o_ref):
    o_ref[...] = lax.dot_general(
        x_ref[...], w_ref[...], (((1,), (1,)), ((), ())),
        preferred_element_type=jnp.float32)


def _mm_nt(x, w):
    return pl.pallas_call(
        _mm_nt_body,
        grid=(GRID,),
        in_specs=[
            pl.BlockSpec((BM, D), lambda i: (i, 0)),
            pl.BlockSpec(w.shape, lambda i: (0, 0)),
        ],
        out_specs=pl.BlockSpec((BM, D), lambda i: (i, 0)),
        out_shape=jax.ShapeDtypeStruct((N, D), jnp.float32),
    )(x, w)


def _zlin_body(x_ref, w_ref, b_ref, o_ref):
    o_ref[...] = lax.dot_general(
        x_ref[...], w_ref[...], (((1,), (1,)), ((), ())),
        preferred_element_type=jnp.float32) + b_ref[...]


def _zlin(x, w, b):
    return pl.pallas_call(
        _zlin_body,
        grid=(GRID,),
        in_specs=[
            pl.BlockSpec((BM, D), lambda i: (i, 0)),
            pl.BlockSpec((D, D), lambda i: (0, 0)),
            pl.BlockSpec((D,), lambda i: (0,)),
        ],
        out_specs=pl.BlockSpec((BM, D), lambda i: (i, 0)),
        out_shape=jax.ShapeDtypeStruct((N, D), jnp.float32),
    )(x, w, b)


def _comb_body(relu, a0_ref, a1_ref, z_ref, wn_ref, h_ref, y_ref):
    h = a0_ref[...] + a1_ref[...] + z_ref[...]
    if relu:
        h = jnp.maximum(h, 0.0)
    h_ref[...] = h
    y_ref[...] = lax.dot_general(h, wn_ref[...], (((1,), (1,)), ((), ())),
                                 preferred_element_type=jnp.float32)


def _comb(a0, a1, z, w_next, relu):
    return pl.pallas_call(
        functools.partial(_comb_body, relu),
        grid=(GRID,),
        in_specs=[
            pl.BlockSpec((BM, D), lambda i: (i, 0)),
            pl.BlockSpec((BM, D), lambda i: (i, 0)),
            pl.BlockSpec((BM, D), lambda i: (i, 0)),
            pl.BlockSpec((D, D), lambda i: (0, 0)),
        ],
        out_specs=[
            pl.BlockSpec((BM, D), lambda i: (i, 0)),
            pl.BlockSpec((BM, D), lambda i: (i, 0)),
        ],
        out_shape=[
            jax.ShapeDtypeStruct((N, D), jnp.float32),
            jax.ShapeDtypeStruct((N, D), jnp.float32),
        ],
    )(a0, a1, z, w_next)


def _final_body(a0_ref, a1_ref, z_ref, bat_ref, wl_ref, bl_ref,
                o_ref, pool_ref, cnt_ref):
    i = pl.program_id(0)

    @pl.when(i == 0)
    def _init():
        pool_ref[...] = jnp.zeros_like(pool_ref)
        cnt_ref[...] = jnp.zeros_like(cnt_ref)

    h = a0_ref[...] + a1_ref[...] + z_ref[...]
    gids = bat_ref[...]
    iot = lax.broadcasted_iota(jnp.int32, (BM, G), 1)
    onehot = jnp.where(gids == iot, 1.0, 0.0)
    pool_ref[...] += lax.dot_general(
        onehot, h, (((0,), (0,)), ((), ())), preferred_element_type=jnp.float32)
    cnt_ref[...] += lax.dot_general(
        onehot, jnp.ones((BM, D), jnp.float32), (((0,), (0,)), ((), ())),
        preferred_element_type=jnp.float32)

    @pl.when(i == GRID - 1)
    def _done():
        pooled = pool_ref[...] / jnp.maximum(cnt_ref[...], 1.0)
        o_ref[...] = lax.dot_general(
            pooled, wl_ref[...], (((1,), (1,)), ((), ())),
            preferred_element_type=jnp.float32) + bl_ref[...]


def _final(a0, a1, z, batch2d, w_lin, b_lin):
    return pl.pallas_call(
        _final_body,
        grid=(GRID,),
        in_specs=[
            pl.BlockSpec((BM, D), lambda i: (i, 0)),
            pl.BlockSpec((BM, D), lambda i: (i, 0)),
            pl.BlockSpec((BM, D), lambda i: (i, 0)),
            pl.BlockSpec((BM, 1), lambda i: (i, 0)),
            pl.BlockSpec((C, D), lambda i: (0, 0)),
            pl.BlockSpec((C,), lambda i: (0,)),
        ],
        out_specs=pl.BlockSpec((G, C), lambda i: (0, 0)),
        out_shape=jax.ShapeDtypeStruct((G, C), jnp.float32),
        scratch_shapes=[
            pltpu.VMEM((G, D), jnp.float32),
            pltpu.VMEM((G, D), jnp.float32),
        ],
    )(a0, a1, z, batch2d, w_lin, b_lin)


def kernel(x, edge_index, batch,
           W1_rel, b1_rel, W1_root,
           W2_rel, b2_rel, W2_root,
           W3_rel, b3_rel, W3_root,
           W_lin, b_lin):
    npad = EP - E
    pad_src = jnp.arange(npad, dtype=jnp.int32) % N
    pad_dst = N + (jnp.arange(npad, dtype=jnp.int32) % (NPAD - N))
    src = jnp.concatenate([edge_index[0], pad_src]).reshape(NW, EW)
    dst = jnp.concatenate([edge_index[1], pad_dst]).reshape(NW, NCHUNK, CH)
    batch2d = batch.reshape(N, 1)

    y1 = _mm_nt(x, W1_rel)
    a1 = _seg_sum(y1, src, dst)
    z1 = _zlin(x, W1_root, b1_rel)
    h1, y2 = _comb(a1[0, :N], a1[1, :N], z1, W2_rel, relu=True)
    a2 = _seg_sum(y2, src, dst)
    z2 = _zlin(h1, W2_root, b2_rel)
    h2, y3 = _comb(a2[0, :N], a2[1, :N], z2, W3_rel, relu=True)
    a3 = _seg_sum(y3, src, dst)
    z3 = _zlin(h2, W3_root, b3_rel)
    return _final(a3[0, :N], a3[1, :N], z3, batch2d, W_lin, b_lin)


# speedup vs baseline: 5.3659x; 1.0145x over previous
import functools

import jax
import jax.numpy as jnp
from jax import lax
from jax.experimental import pallas as pl
from jax.experimental.pallas import tpu as pltpu
from jax.experimental.pallas import tpu_sc as plsc

N = 10000
E = 320000
D = 128
G = 64
C = 10

NC = 2
NS = 16
NW = NC * NS
CH = 80
NCHUNK = 125
EW = NCHUNK * CH
NPAD = NS * 632
RPT = NPAD // NS
MSG_BYTES = CH * D * 4

BM = 2000
GRID = N // BM

_mesh = plsc.VectorSubcoreMesh(
    core_axis_name="c", subcore_axis_name="s", num_cores=NC, num_subcores=NS)


@functools.partial(
    pl.kernel,
    out_type=jax.ShapeDtypeStruct((NC, NPAD, D), jnp.float32),
    mesh=_mesh,
    scratch_types=[
        pltpu.VMEM((EW,), jnp.int32),
        pltpu.VMEM((NCHUNK, CH), jnp.int32),
        [pltpu.VMEM((CH, D), jnp.float32) for _ in range(2)],
        [pltpu.SemaphoreType.DMA for _ in range(2)],
        [pltpu.SemaphoreType.DMA for _ in range(2)],
        pltpu.VMEM_SHARED((NPAD, D), jnp.float32),
    ],
)
def _seg_sum(y_hbm, src_hbm, dst_hbm, out_hbm, src_v, dst_v, msgs, gsems, ssems,
             acc_sh):
    c = lax.axis_index("c")
    s = lax.axis_index("s")
    wid = s * NC + c

    zeros16 = jnp.zeros((16,), jnp.float32)

    def _zero_row(r, carry):
        for j in range(D // 16):
            msgs[0][r, pl.ds(j * 16, 16)] = zeros16
        return carry

    lax.fori_loop(0, CH, _zero_row, 0)
    row0 = s * RPT
    for k in range(RPT // CH):
        pltpu.sync_copy(msgs[0], acc_sh.at[pl.ds(row0 + k * CH, CH)])
    pltpu.sync_copy(msgs[0].at[pl.ds(0, RPT % CH)],
                    acc_sh.at[pl.ds(row0 + RPT - RPT % CH, RPT % CH)])
    plsc.subcore_barrier()

    pltpu.sync_copy(src_hbm.at[wid], src_v)
    pltpu.sync_copy(dst_hbm.at[wid], dst_v)

    def _fire_gather(j, b):
        base = pl.multiple_of(j * CH, CH)
        pltpu.async_copy(y_hbm.at[src_v.at[pl.ds(base, CH)]], msgs[b], gsems[b])

    def _sem_wait(sem, b):
        pltpu.make_async_copy(y_hbm.at[pl.ds(0, CH)], msgs[b], sem).wait()

    def _visit(j, b, mode):
        if mode != 2:
            if mode == 1:
                _sem_wait(ssems[1 - b], 1 - b)
            _fire_gather(j + 1, 1 - b)
        _sem_wait(gsems[b], b)
        pltpu.async_copy(msgs[b], acc_sh.at[dst_v.at[j]], ssems[b], add=True)

    _fire_gather(0, 0)
    _visit(0, 0, 0)

    def _ring(it, carry):
        j0 = 1 + 2 * it
        _visit(j0, 1, 1)
        _visit(j0 + 1, 0, 1)
        return carry

    lax.fori_loop(0, (NCHUNK - 2) // 2, _ring, 0)
    _visit(NCHUNK - 1, 1, 2)
    _sem_wait(ssems[0], 0)
    _sem_wait(ssems[1], 1)
    plsc.subcore_barrier()

    pltpu.sync_copy(acc_sh.at[pl.ds(row0, RPT)], out_hbm.at[c, pl.ds(row0, RPT)])


def _mm_nt_body(x_ref, w_ref, o_ref):
    o_ref[...] = lax.dot_general(
        x_ref[...], w_ref[...], (((1,), (1,)), ((), ())),
        preferred_element_type=jnp.float32)


def _mm_nt(x, w):
    return pl.pallas_call(
        _mm_nt_body,
        grid=(GRID,),
        in_specs=[
            pl.BlockSpec((BM, D), lambda i: (i, 0)),
            pl.BlockSpec(w.shape, lambda i: (0, 0)),
        ],
        out_specs=pl.BlockSpec((BM, D), lambda i: (i, 0)),
        out_shape=jax.ShapeDtypeStruct((N, D), jnp.float32),
    )(x, w)


def _zlin_body(x_ref, w_ref, b_ref, o_ref):
    o_ref[...] = lax.dot_general(
        x_ref[...], w_ref[...], (((1,), (1,)), ((), ())),
        preferred_element_type=jnp.float32) + b_ref[...]


def _zlin(x, w, b):
    return pl.pallas_call(
        _zlin_body,
        grid=(GRID,),
        in_specs=[
            pl.BlockSpec((BM, D), lambda i: (i, 0)),
            pl.BlockSpec((D, D), lambda i: (0, 0)),
            pl.BlockSpec((D,), lambda i: (0,)),
        ],
        out_specs=pl.BlockSpec((BM, D), lambda i: (i, 0)),
        out_shape=jax.ShapeDtypeStruct((N, D), jnp.float32),
    )(x, w, b)


def _comb_body(relu, a_ref0, a_ref1, z_ref, wn_ref, h_ref, y_ref):
    h = a_ref0[0] + a_ref1[0] + z_ref[...]
    if relu:
        h = jnp.maximum(h, 0.0)
    h_ref[...] = h
    y_ref[...] = lax.dot_general(h, wn_ref[...], (((1,), (1,)), ((), ())),
                                 preferred_element_type=jnp.float32)


def _comb(a, z, w_next, relu):
    return pl.pallas_call(
        functools.partial(_comb_body, relu),
        grid=(GRID,),
        in_specs=[
            pl.BlockSpec((1, BM, D), lambda i: (0, i, 0)),
            pl.BlockSpec((1, BM, D), lambda i: (1, i, 0)),
            pl.BlockSpec((BM, D), lambda i: (i, 0)),
            pl.BlockSpec((D, D), lambda i: (0, 0)),
        ],
        out_specs=[
            pl.BlockSpec((BM, D), lambda i: (i, 0)),
            pl.BlockSpec((BM, D), lambda i: (i, 0)),
        ],
        out_shape=[
            jax.ShapeDtypeStruct((N, D), jnp.float32),
            jax.ShapeDtypeStruct((N, D), jnp.float32),
        ],
    )(a, a, z, w_next)


def _final_body(a_ref0, a_ref1, z_ref, bat_ref, wl_ref, bl_ref,
                o_ref, pool_ref, cnt_ref):
    i = pl.program_id(0)

    @pl.when(i == 0)
    def _init():
        pool_ref[...] = jnp.zeros_like(pool_ref)
        cnt_ref[...] = jnp.zeros_like(cnt_ref)

    h = a_ref0[0] + a_ref1[0] + z_ref[...]
    gids = bat_ref[...]
    iot = lax.broadcasted_iota(jnp.int32, (BM, G), 1)
    onehot = jnp.where(gids == iot, 1.0, 0.0)
    pool_ref[...] += lax.dot_general(
        onehot, h, (((0,), (0,)), ((), ())), preferred_element_type=jnp.float32)
    cnt_ref[...] += lax.dot_general(
        onehot, jnp.ones((BM, D), jnp.float32), (((0,), (0,)), ((), ())),
        preferred_element_type=jnp.float32)

    @pl.when(i == GRID - 1)
    def _done():
        pooled = pool_ref[...] / jnp.maximum(cnt_ref[...], 1.0)
        o_ref[...] = lax.dot_general(
            pooled, wl_ref[...], (((1,), (1,)), ((), ())),
            preferred_element_type=jnp.float32) + bl_ref[...]


def _final(a, z, batch2d, w_lin, b_lin):
    return pl.pallas_call(
        _final_body,
        grid=(GRID,),
        in_specs=[
            pl.BlockSpec((1, BM, D), lambda i: (0, i, 0)),
            pl.BlockSpec((1, BM, D), lambda i: (1, i, 0)),
            pl.BlockSpec((BM, D), lambda i: (i, 0)),
            pl.BlockSpec((BM, 1), lambda i: (i, 0)),
            pl.BlockSpec((C, D), lambda i: (0, 0)),
            pl.BlockSpec((C,), lambda i: (0,)),
        ],
        out_specs=pl.BlockSpec((G, C), lambda i: (0, 0)),
        out_shape=jax.ShapeDtypeStruct((G, C), jnp.float32),
        scratch_shapes=[
            pltpu.VMEM((G, D), jnp.float32),
            pltpu.VMEM((G, D), jnp.float32),
        ],
    )(a, a, z, batch2d, w_lin, b_lin)


def kernel(x, edge_index, batch,
           W1_rel, b1_rel, W1_root,
           W2_rel, b2_rel, W2_root,
           W3_rel, b3_rel, W3_root,
           W_lin, b_lin):
    src = edge_index[0].reshape(NW, EW)
    dst = edge_index[1].reshape(NW, NCHUNK, CH)
    batch2d = batch.reshape(N, 1)

    y1 = _mm_nt(x, W1_rel)
    a1 = _seg_sum(y1, src, dst)
    z1 = _zlin(x, W1_root, b1_rel)
    h1, y2 = _comb(a1, z1, W2_rel, relu=True)
    a2 = _seg_sum(y2, src, dst)
    z2 = _zlin(h1, W2_root, b2_rel)
    h2, y3 = _comb(a2, z2, W3_rel, relu=True)
    a3 = _seg_sum(y3, src, dst)
    z3 = _zlin(h2, W3_root, b3_rel)
    return _final(a3, z3, batch2d, W_lin, b_lin)


# speedup vs baseline: 5.4694x; 1.0193x over previous
import functools

import jax
import jax.numpy as jnp
from jax import lax
from jax.experimental import pallas as pl
from jax.experimental.pallas import tpu as pltpu
from jax.experimental.pallas import tpu_sc as plsc

N = 10000
E = 320000
D = 128
G = 64
C = 10

NC = 2
NS = 16
NW = NC * NS
CH = 80
NCHUNK = 125
EW = NCHUNK * CH
NPAD = NS * 632
RPT = NPAD // NS
MSG_BYTES = CH * D * 4

BM = 2000
GRID = N // BM

_mesh = plsc.VectorSubcoreMesh(
    core_axis_name="c", subcore_axis_name="s", num_cores=NC, num_subcores=NS)


@functools.partial(
    pl.kernel,
    out_type=jax.ShapeDtypeStruct((NC, NPAD, D), jnp.float32),
    mesh=_mesh,
    scratch_types=[
        pltpu.VMEM((EW,), jnp.int32),
        pltpu.VMEM((NCHUNK, CH), jnp.int32),
        [pltpu.VMEM((CH, D), jnp.float32) for _ in range(2)],
        [pltpu.SemaphoreType.DMA for _ in range(2)],
        [pltpu.SemaphoreType.DMA for _ in range(2)],
        pltpu.VMEM_SHARED((NPAD, D), jnp.float32),
    ],
)
def _seg_sum(y_hbm, src_hbm, dst_hbm, out_hbm, src_v, dst_v, msgs, gsems, ssems,
             acc_sh):
    c = lax.axis_index("c")
    s = lax.axis_index("s")
    wid = s * NC + c

    zeros16 = jnp.zeros((16,), jnp.float32)

    def _zero_row(r, carry):
        for j in range(D // 16):
            msgs[0][r, pl.ds(j * 16, 16)] = zeros16
        return carry

    lax.fori_loop(0, CH, _zero_row, 0)
    idx_copy = pltpu.async_copy(src_hbm.at[wid], src_v, gsems[0])
    idx_copy2 = pltpu.async_copy(dst_hbm.at[wid], dst_v, gsems[1])
    row0 = s * RPT
    zcopies = []
    for k in range(RPT // CH):
        zcopies.append(pltpu.async_copy(
            msgs[0], acc_sh.at[pl.ds(row0 + k * CH, CH)], ssems[0]))
    zcopies.append(pltpu.async_copy(
        msgs[0].at[pl.ds(0, RPT % CH)],
        acc_sh.at[pl.ds(row0 + RPT - RPT % CH, RPT % CH)], ssems[1]))
    idx_copy.wait()
    idx_copy2.wait()
    for cp in zcopies:
        cp.wait()
    plsc.subcore_barrier()

    def _fire_gather(j, b):
        base = pl.multiple_of(j * CH, CH)
        pltpu.async_copy(y_hbm.at[src_v.at[pl.ds(base, CH)]], msgs[b], gsems[b])

    def _sem_wait(sem, b):
        pltpu.make_async_copy(y_hbm.at[pl.ds(0, CH)], msgs[b], sem).wait()

    def _visit(j, b, mode):
        if mode != 2:
            if mode == 1:
                _sem_wait(ssems[1 - b], 1 - b)
            _fire_gather(j + 1, 1 - b)
        _sem_wait(gsems[b], b)
        pltpu.async_copy(msgs[b], acc_sh.at[dst_v.at[j]], ssems[b], add=True)

    _fire_gather(0, 0)
    _visit(0, 0, 0)

    def _ring(it, carry):
        j0 = 1 + 2 * it
        _visit(j0, 1, 1)
        _visit(j0 + 1, 0, 1)
        return carry

    lax.fori_loop(0, (NCHUNK - 2) // 2, _ring, 0)
    _visit(NCHUNK - 1, 1, 2)
    _sem_wait(ssems[0], 0)
    _sem_wait(ssems[1], 1)
    plsc.subcore_barrier()

    pltpu.sync_copy(acc_sh.at[pl.ds(row0, RPT)], out_hbm.at[c, pl.ds(row0, RPT)])


def _mm_nt_body(x_ref, w_ref, o_ref):
    o_ref[...] = lax.dot_general(
        x_ref[...], w_ref[...], (((1,), (1,)), ((), ())),
        preferred_element_type=jnp.float32)


def _mm_nt(x, w):
    return pl.pallas_call(
        _mm_nt_body,
        grid=(GRID,),
        in_specs=[
            pl.BlockSpec((BM, D), lambda i: (i, 0)),
            pl.BlockSpec(w.shape, lambda i: (0, 0)),
        ],
        out_specs=pl.BlockSpec((BM, D), lambda i: (i, 0)),
        out_shape=jax.ShapeDtypeStruct((N, D), jnp.float32),
    )(x, w)


def _zlin_body(x_ref, w_ref, b_ref, o_ref):
    o_ref[...] = lax.dot_general(
        x_ref[...], w_ref[...], (((1,), (1,)), ((), ())),
        preferred_element_type=jnp.float32) + b_ref[...]


def _zlin(x, w, b):
    return pl.pallas_call(
        _zlin_body,
        grid=(GRID,),
        in_specs=[
            pl.BlockSpec((BM, D), lambda i: (i, 0)),
            pl.BlockSpec((D, D), lambda i: (0, 0)),
            pl.BlockSpec((D,), lambda i: (0,)),
        ],
        out_specs=pl.BlockSpec((BM, D), lambda i: (i, 0)),
        out_shape=jax.ShapeDtypeStruct((N, D), jnp.float32),
    )(x, w, b)


def _comb_body(relu, a_ref0, a_ref1, z_ref, wn_ref, h_ref, y_ref):
    h = a_ref0[0] + a_ref1[0] + z_ref[...]
    if relu:
        h = jnp.maximum(h, 0.0)
    h_ref[...] = h
    y_ref[...] = lax.dot_general(h, wn_ref[...], (((1,), (1,)), ((), ())),
                                 preferred_element_type=jnp.float32)


def _comb(a, z, w_next, relu):
    return pl.pallas_call(
        functools.partial(_comb_body, relu),
        grid=(GRID,),
        in_specs=[
            pl.BlockSpec((1, BM, D), lambda i: (0, i, 0)),
            pl.BlockSpec((1, BM, D), lambda i: (1, i, 0)),
            pl.BlockSpec((BM, D), lambda i: (i, 0)),
            pl.BlockSpec((D, D), lambda i: (0, 0)),
        ],
        out_specs=[
            pl.BlockSpec((BM, D), lambda i: (i, 0)),
            pl.BlockSpec((BM, D), lambda i: (i, 0)),
        ],
        out_shape=[
            jax.ShapeDtypeStruct((N, D), jnp.float32),
            jax.ShapeDtypeStruct((N, D), jnp.float32),
        ],
    )(a, a, z, w_next)


def _final_body(a_ref0, a_ref1, z_ref, bat_ref, wl_ref, bl_ref,
                o_ref, pool_ref, cnt_ref):
    i = pl.program_id(0)

    @pl.when(i == 0)
    def _init():
        pool_ref[...] = jnp.zeros_like(pool_ref)
        cnt_ref[...] = jnp.zeros_like(cnt_ref)

    h = a_ref0[0] + a_ref1[0] + z_ref[...]
    gids = bat_ref[...]
    iot = lax.broadcasted_iota(jnp.int32, (BM, G), 1)
    onehot = jnp.where(gids == iot, 1.0, 0.0)
    pool_ref[...] += lax.dot_general(
        onehot, h, (((0,), (0,)), ((), ())), preferred_element_type=jnp.float32)
    cnt_ref[...] += lax.dot_general(
        onehot, jnp.ones((BM, D), jnp.float32), (((0,), (0,)), ((), ())),
        preferred_element_type=jnp.float32)

    @pl.when(i == GRID - 1)
    def _done():
        pooled = pool_ref[...] / jnp.maximum(cnt_ref[...], 1.0)
        o_ref[...] = lax.dot_general(
            pooled, wl_ref[...], (((1,), (1,)), ((), ())),
            preferred_element_type=jnp.float32) + bl_ref[...]


def _final(a, z, batch2d, w_lin, b_lin):
    return pl.pallas_call(
        _final_body,
        grid=(GRID,),
        in_specs=[
            pl.BlockSpec((1, BM, D), lambda i: (0, i, 0)),
            pl.BlockSpec((1, BM, D), lambda i: (1, i, 0)),
            pl.BlockSpec((BM, D), lambda i: (i, 0)),
            pl.BlockSpec((BM, 1), lambda i: (i, 0)),
            pl.BlockSpec((C, D), lambda i: (0, 0)),
            pl.BlockSpec((C,), lambda i: (0,)),
        ],
        out_specs=pl.BlockSpec((G, C), lambda i: (0, 0)),
        out_shape=jax.ShapeDtypeStruct((G, C), jnp.float32),
        scratch_shapes=[
            pltpu.VMEM((G, D), jnp.float32),
            pltpu.VMEM((G, D), jnp.float32),
        ],
    )(a, a, z, batch2d, w_lin, b_lin)


def kernel(x, edge_index, batch,
           W1_rel, b1_rel, W1_root,
           W2_rel, b2_rel, W2_root,
           W3_rel, b3_rel, W3_root,
           W_lin, b_lin):
    src = edge_index[0].reshape(NW, EW)
    dst = edge_index[1].reshape(NW, NCHUNK, CH)
    batch2d = batch.reshape(N, 1)

    y1 = _mm_nt(x, W1_rel)
    a1 = _seg_sum(y1, src, dst)
    z1 = _zlin(x, W1_root, b1_rel)
    h1, y2 = _comb(a1, z1, W2_rel, relu=True)
    a2 = _seg_sum(y2, src, dst)
    z2 = _zlin(h1, W2_root, b2_rel)
    h2, y3 = _comb(a2, z2, W3_rel, relu=True)
    a3 = _seg_sum(y3, src, dst)
    z3 = _zlin(h2, W3_root, b3_rel)
    return _final(a3, z3, batch2d, W_lin, b_lin)


# speedup vs baseline: 5.4912x; 1.0040x over previous
import functools

import jax
import jax.numpy as jnp
from jax import lax
from jax.experimental import pallas as pl
from jax.experimental.pallas import tpu as pltpu
from jax.experimental.pallas import tpu_sc as plsc

N = 10000
E = 320000
D = 128
G = 64
C = 10

NC = 2
NS = 16
NW = NC * NS
CH = 80
NCHUNK = 125
EW = NCHUNK * CH
NPAD = NS * 632
RPT = NPAD // NS
MSG_BYTES = CH * D * 4

BM = 2000
GRID = N // BM

_mesh = plsc.VectorSubcoreMesh(
    core_axis_name="c", subcore_axis_name="s", num_cores=NC, num_subcores=NS)


@functools.partial(
    pl.kernel,
    out_type=jax.ShapeDtypeStruct((NC, NPAD, D), jnp.float32),
    mesh=_mesh,
    scratch_types=[
        pltpu.VMEM((EW,), jnp.int32),
        pltpu.VMEM((NCHUNK, CH), jnp.int32),
        [pltpu.VMEM((CH, D), jnp.float32) for _ in range(2)],
        [pltpu.SemaphoreType.DMA for _ in range(2)],
        [pltpu.SemaphoreType.DMA for _ in range(2)],
        pltpu.VMEM_SHARED((NPAD, D), jnp.float32),
    ],
)
def _seg_sum(y_hbm, src_hbm, dst_hbm, out_hbm, src_v, dst_v, msgs, gsems, ssems,
             acc_sh):
    c = lax.axis_index("c")
    s = lax.axis_index("s")
    wid = s * NC + c

    zeros16 = jnp.zeros((16,), jnp.float32)

    def _zero_row(r, carry):
        for j in range(D // 16):
            msgs[0][r, pl.ds(j * 16, 16)] = zeros16
        return carry

    lax.fori_loop(0, CH, _zero_row, 0)
    idx_copy = pltpu.async_copy(src_hbm.at[wid], src_v, gsems[0])
    idx_copy2 = pltpu.async_copy(dst_hbm.at[wid], dst_v, gsems[1])
    row0 = s * RPT
    zcopies = []
    for k in range(RPT // CH):
        zcopies.append(pltpu.async_copy(
            msgs[0], acc_sh.at[pl.ds(row0 + k * CH, CH)], ssems[0]))
    zcopies.append(pltpu.async_copy(
        msgs[0].at[pl.ds(0, RPT % CH)],
        acc_sh.at[pl.ds(row0 + RPT - RPT % CH, RPT % CH)], ssems[1]))
    idx_copy.wait()
    idx_copy2.wait()
    for cp in zcopies:
        cp.wait()
    plsc.subcore_barrier()

    def _fire_gather(j, b):
        base = pl.multiple_of(j * CH, CH)
        pltpu.async_copy(y_hbm.at[src_v.at[pl.ds(base, CH)]], msgs[b], gsems[b])

    def _sem_wait(sem, b):
        pltpu.make_async_copy(y_hbm.at[pl.ds(0, CH)], msgs[b], sem).wait()

    def _visit(j, b, mode):
        if mode != 2:
            if mode == 1:
                _sem_wait(ssems[1 - b], 1 - b)
            _fire_gather(j + 1, 1 - b)
        _sem_wait(gsems[b], b)
        pltpu.async_copy(msgs[b], acc_sh.at[dst_v.at[j]], ssems[b], add=True)

    _fire_gather(0, 0)
    _visit(0, 0, 0)

    def _ring(it, carry):
        j0 = 1 + 2 * it
        _visit(j0, 1, 1)
        _visit(j0 + 1, 0, 1)
        return carry

    lax.fori_loop(0, (NCHUNK - 2) // 2, _ring, 0)
    _visit(NCHUNK - 1, 1, 2)
    _sem_wait(ssems[0], 0)
    _sem_wait(ssems[1], 1)
    plsc.subcore_barrier()

    pltpu.sync_copy(acc_sh.at[pl.ds(row0, RPT)], out_hbm.at[c, pl.ds(row0, RPT)])


def _mm_nt_body(x_ref, w_ref, o_ref):
    o_ref[...] = lax.dot_general(
        x_ref[...], w_ref[...], (((1,), (1,)), ((), ())),
        preferred_element_type=jnp.float32)


def _mm_nt(x, w):
    return pl.pallas_call(
        _mm_nt_body,
        grid=(GRID,),
        in_specs=[
            pl.BlockSpec((BM, D), lambda i: (i, 0)),
            pl.BlockSpec(w.shape, lambda i: (0, 0)),
        ],
        out_specs=pl.BlockSpec((BM, D), lambda i: (i, 0)),
        out_shape=jax.ShapeDtypeStruct((N, D), jnp.float32),
    )(x, w)


def _zlin_body(x_ref, w_ref, b_ref, o_ref):
    o_ref[...] = lax.dot_general(
        x_ref[...], w_ref[...], (((1,), (1,)), ((), ())),
        preferred_element_type=jnp.float32) + b_ref[...]


def _zlin(x, w, b):
    return pl.pallas_call(
        _zlin_body,
        grid=(GRID,),
        in_specs=[
            pl.BlockSpec((BM, D), lambda i: (i, 0)),
            pl.BlockSpec((D, D), lambda i: (0, 0)),
            pl.BlockSpec((D,), lambda i: (0,)),
        ],
        out_specs=pl.BlockSpec((BM, D), lambda i: (i, 0)),
        out_shape=jax.ShapeDtypeStruct((N, D), jnp.float32),
    )(x, w, b)


def _comb_body(a_ref0, a_ref1, z_ref, wn_ref, y_ref):
    h = jnp.maximum(a_ref0[0] + a_ref1[0] + z_ref[...], 0.0)
    y_ref[...] = lax.dot_general(h, wn_ref[...], (((1,), (1,)), ((), ())),
                                 preferred_element_type=jnp.float32)


def _comb(a, z, w_next):
    return pl.pallas_call(
        _comb_body,
        grid=(GRID,),
        in_specs=[
            pl.BlockSpec((1, BM, D), lambda i: (0, i, 0)),
            pl.BlockSpec((1, BM, D), lambda i: (1, i, 0)),
            pl.BlockSpec((BM, D), lambda i: (i, 0)),
            pl.BlockSpec((D, D), lambda i: (0, 0)),
        ],
        out_specs=pl.BlockSpec((BM, D), lambda i: (i, 0)),
        out_shape=jax.ShapeDtypeStruct((N, D), jnp.float32),
    )(a, a, z, w_next)


def _zlin2_body(a_ref0, a_ref1, z_ref, w_ref, b_ref, o_ref):
    h = jnp.maximum(a_ref0[0] + a_ref1[0] + z_ref[...], 0.0)
    o_ref[...] = lax.dot_general(
        h, w_ref[...], (((1,), (1,)), ((), ())),
        preferred_element_type=jnp.float32) + b_ref[...]


def _zlin2(a, z, w, b):
    return pl.pallas_call(
        _zlin2_body,
        grid=(GRID,),
        in_specs=[
            pl.BlockSpec((1, BM, D), lambda i: (0, i, 0)),
            pl.BlockSpec((1, BM, D), lambda i: (1, i, 0)),
            pl.BlockSpec((BM, D), lambda i: (i, 0)),
            pl.BlockSpec((D, D), lambda i: (0, 0)),
            pl.BlockSpec((D,), lambda i: (0,)),
        ],
        out_specs=pl.BlockSpec((BM, D), lambda i: (i, 0)),
        out_shape=jax.ShapeDtypeStruct((N, D), jnp.float32),
    )(a, a, z, w, b)


def _final_body(a_ref0, a_ref1, z_ref, bat_ref, wl_ref, bl_ref,
                o_ref, pool_ref, cnt_ref):
    i = pl.program_id(0)

    @pl.when(i == 0)
    def _init():
        pool_ref[...] = jnp.zeros_like(pool_ref)
        cnt_ref[...] = jnp.zeros_like(cnt_ref)

    h = a_ref0[0] + a_ref1[0] + z_ref[...]
    gids = bat_ref[...]
    iot = lax.broadcasted_iota(jnp.int32, (BM, G), 1)
    onehot = jnp.where(gids == iot, 1.0, 0.0)
    pool_ref[...] += lax.dot_general(
        onehot, h, (((0,), (0,)), ((), ())), preferred_element_type=jnp.float32)
    cnt_ref[...] += lax.dot_general(
        onehot, jnp.ones((BM, D), jnp.float32), (((0,), (0,)), ((), ())),
        preferred_element_type=jnp.float32)

    @pl.when(i == GRID - 1)
    def _done():
        pooled = pool_ref[...] / jnp.maximum(cnt_ref[...], 1.0)
        o_ref[...] = lax.dot_general(
            pooled, wl_ref[...], (((1,), (1,)), ((), ())),
            preferred_element_type=jnp.float32) + bl_ref[...]


def _final(a, z, batch2d, w_lin, b_lin):
    return pl.pallas_call(
        _final_body,
        grid=(GRID,),
        in_specs=[
            pl.BlockSpec((1, BM, D), lambda i: (0, i, 0)),
            pl.BlockSpec((1, BM, D), lambda i: (1, i, 0)),
            pl.BlockSpec((BM, D), lambda i: (i, 0)),
            pl.BlockSpec((BM, 1), lambda i: (i, 0)),
            pl.BlockSpec((C, D), lambda i: (0, 0)),
            pl.BlockSpec((C,), lambda i: (0,)),
        ],
        out_specs=pl.BlockSpec((G, C), lambda i: (0, 0)),
        out_shape=jax.ShapeDtypeStruct((G, C), jnp.float32),
        scratch_shapes=[
            pltpu.VMEM((G, D), jnp.float32),
            pltpu.VMEM((G, D), jnp.float32),
        ],
    )(a, a, z, batch2d, w_lin, b_lin)


def kernel(x, edge_index, batch,
           W1_rel, b1_rel, W1_root,
           W2_rel, b2_rel, W2_root,
           W3_rel, b3_rel, W3_root,
           W_lin, b_lin):
    src = edge_index[0].reshape(NW, EW)
    dst = edge_index[1].reshape(NW, NCHUNK, CH)
    batch2d = batch.reshape(N, 1)

    y1 = _mm_nt(x, W1_rel)
    a1 = _seg_sum(y1, src, dst)
    z1 = _zlin(x, W1_root, b1_rel)
    y2 = _comb(a1, z1, W2_rel)
    a2 = _seg_sum(y2, src, dst)
    z2 = _zlin2(a1, z1, W2_root, b2_rel)
    y3 = _comb(a2, z2, W3_rel)
    a3 = _seg_sum(y3, src, dst)
    z3 = _zlin2(a2, z2, W3_root, b3_rel)
    return _final(a3, z3, batch2d, W_lin, b_lin)
